# baseline (speedup 1.0000x reference)
import functools

import jax
import jax.numpy as jnp
from jax import lax
from jax.experimental import pallas as pl
from jax.experimental.pallas import tpu as pltpu
from jax.experimental.pallas import tpu_sc as plsc

B = 16384
V = 100000
D_ART = 64
EPS = 1e-3

_NC = 2
_NS = 16
NW = _NC * _NS
BPW = B // NW
IDX_CHUNK = 128
NCHUNK = BPW // IDX_CHUNK


_sc_mesh = plsc.VectorSubcoreMesh(
    core_axis_name="c", subcore_axis_name="s", num_cores=_NC, num_subcores=_NS)


@functools.partial(
    pl.kernel,
    out_type=(
        jax.ShapeDtypeStruct((B, D_ART), jnp.float32),
        jax.ShapeDtypeStruct((B, 1), jnp.int32),
        jax.ShapeDtypeStruct((B, 1), jnp.int32),
        jax.ShapeDtypeStruct((B, 1), jnp.int32),
        jax.ShapeDtypeStruct((B, 1), jnp.int32),
    ),
    mesh=_sc_mesh,
    scratch_types=(
        pltpu.VMEM((NCHUNK, IDX_CHUNK), jnp.int32),
        pltpu.VMEM((BPW, D_ART), jnp.float32),
        pltpu.VMEM((BPW, 1), jnp.int32),
        pltpu.VMEM((BPW, 1), jnp.int32),
        pltpu.VMEM((BPW, 1), jnp.int32),
        pltpu.VMEM((BPW, 1), jnp.int32),
        pltpu.SemaphoreType.DMA,
    ),
    compiler_params=pltpu.CompilerParams(use_tc_tiling_on_sc=False),
)
def _sc_gather(ids_hbm, emb_hbm, smap_hbm, gmap_hbm, grmap_hbm, cmap_hbm,
               art_out, sec_out, grp_out, gra_out, col_out,
               idx_v, rows_v, s_v, g_v, gr_v, c_v, sem):
    wid = lax.axis_index("s") * _NC + lax.axis_index("c")
    base = wid * BPW
    pltpu.sync_copy(ids_hbm.at[pl.ds(wid * NCHUNK, NCHUNK)], idx_v)
    copies = []
    for j in range(NCHUNK):
        idx_j = idx_v.at[j]
        sl = pl.ds(j * IDX_CHUNK, IDX_CHUNK)
        copies.append(pltpu.async_copy(emb_hbm.at[idx_j], rows_v.at[sl], sem))
        copies.append(pltpu.async_copy(smap_hbm.at[idx_j], s_v.at[sl], sem))
        copies.append(pltpu.async_copy(gmap_hbm.at[idx_j], g_v.at[sl], sem))
        copies.append(pltpu.async_copy(grmap_hbm.at[idx_j], gr_v.at[sl], sem))
        copies.append(pltpu.async_copy(cmap_hbm.at[idx_j], c_v.at[sl], sem))
    for cp in copies:
        cp.wait()
    out_sl = pl.ds(base, BPW)
    pltpu.sync_copy(rows_v, art_out.at[out_sl])
    pltpu.sync_copy(s_v, sec_out.at[out_sl])
    pltpu.sync_copy(g_v, grp_out.at[out_sl])
    pltpu.sync_copy(gr_v, gra_out.at[out_sl])
    pltpu.sync_copy(c_v, col_out.at[out_sl])


BLK = 2048


def _tc_body(art_ref, sec_ref, grp_ref, gra_ref, col_ref,
             semb_ref, gemb_ref, gremb_ref, cemb_ref,
             gamma_ref, beta_ref, mean_ref, var_ref, w_ref, out_ref):
    scale = gamma_ref[:] * lax.rsqrt(var_ref[:] + EPS)
    shift = beta_ref[:] - mean_ref[:] * scale

    def onehot_feat(idx_ref, emb_ref, ncls):
        oh = (idx_ref[:] == lax.broadcasted_iota(jnp.int32, (1, ncls), 1))
        return jnp.dot(oh.astype(jnp.float32), emb_ref[:],
                       preferred_element_type=jnp.float32)

    xg = onehot_feat(grp_ref, gemb_ref, 32)
    xgr = onehot_feat(gra_ref, gremb_ref, 32)
    xc = onehot_feat(col_ref, cemb_ref, 32)
    xs = onehot_feat(sec_ref, semb_ref, 64)
    x = jnp.concatenate([art_ref[:], xg, xgr, xc, xs], axis=1)
    x = x * scale + shift
    out_ref[:] = jnp.dot(x, w_ref[:], preferred_element_type=jnp.float32)


def _tc_dense(art, sec, grp, gra, col, semb, gemb, gremb, cemb,
              gamma, beta, mean, var, w):
    grid = (B // BLK,)
    row_blk = lambda width: pl.BlockSpec((BLK, width), lambda i: (i, 0))
    full = lambda a: pl.BlockSpec(a.shape, lambda i: tuple(0 for _ in a.shape))
    return pl.pallas_call(
        _tc_body,
        grid=grid,
        in_specs=[
            row_blk(D_ART), row_blk(1), row_blk(1), row_blk(1), row_blk(1),
            full(semb), full(gemb), full(gremb), full(cemb),
            full(gamma), full(beta), full(mean), full(var), full(w),
        ],
        out_specs=row_blk(128),
        out_shape=jax.ShapeDtypeStruct((B, 128), jnp.float32),
    )(art, sec, grp, gra, col, semb, gemb, gremb, cemb,
      gamma, beta, mean, var, w)


def kernel(article_id, article_emb, section_map, section_emb, group_map,
           group_emb, graphical_map, graphical_emb, colour_map, colour_emb,
           gamma, beta, moving_mean, moving_var, W):
    ids = article_id.astype(jnp.int32).reshape(B // IDX_CHUNK, IDX_CHUNK)
    art, sec, grp, gra, col = _sc_gather(
        ids, article_emb,
        section_map.reshape(V, 1), group_map.reshape(V, 1),
        graphical_map.reshape(V, 1), colour_map.reshape(V, 1))
    ids1 = article_id.astype(jnp.int32)
    sec = jnp.take(section_map, ids1).reshape(B, 1)
    grp = jnp.take(group_map, ids1).reshape(B, 1)
    gra = jnp.take(graphical_map, ids1).reshape(B, 1)
    col = jnp.take(colour_map, ids1).reshape(B, 1)
    return _tc_dense(
        art, sec, grp, gra, col,
        section_emb, group_emb, graphical_emb, colour_emb,
        gamma.reshape(1, 128), beta.reshape(1, 128),
        moving_mean.reshape(1, 128), moving_var.reshape(1, 128), W)


# speedup vs baseline: 4.4538x; 4.4538x over previous
import functools

import jax
import jax.numpy as jnp
from jax import lax
from jax.experimental import pallas as pl
from jax.experimental.pallas import tpu as pltpu
from jax.experimental.pallas import tpu_sc as plsc

B = 16384
V = 100000
D_ART = 64
EPS = 1e-3
LANES = 16

_NC = 2
_NS = 16
NW = _NC * _NS
BPW = B // NW
IDX_CHUNK = 128
NCHUNK = BPW // IDX_CHUNK
NVREG = BPW // LANES

_sc_mesh = plsc.VectorSubcoreMesh(
    core_axis_name="c", subcore_axis_name="s", num_cores=_NC, num_subcores=_NS)


@functools.partial(
    pl.kernel,
    out_type=jax.ShapeDtypeStruct((B, 128), jnp.float32),
    mesh=_sc_mesh,
    scratch_types=(
        pltpu.VMEM((NCHUNK, IDX_CHUNK), jnp.int32),
        pltpu.VMEM((NCHUNK, IDX_CHUNK), jnp.int32),
        pltpu.VMEM((BPW, D_ART), jnp.float32),
        pltpu.VMEM((BPW, LANES), jnp.int32),
        pltpu.VMEM((BPW, LANES), jnp.int32),
        pltpu.VMEM((BPW, LANES), jnp.int32),
        pltpu.VMEM((BPW, LANES), jnp.int32),
        pltpu.VMEM((BPW, 4), jnp.float32),
        pltpu.SemaphoreType.DMA,
    ),
    compiler_params=pltpu.CompilerParams(
        use_tc_tiling_on_sc=False, needs_layout_passes=False),
)
def _sc_gather(ids_hbm, emb_hbm, smap_hbm, gmap_hbm, grmap_hbm, cmap_hbm,
               out_hbm,
               idx_v, idx16_v, rows_v, ms_v, mg_v, mgr_v, mc_v, sidx_v, sem):
    wid = lax.axis_index("s") * _NC + lax.axis_index("c")
    base = wid * BPW
    pltpu.sync_copy(ids_hbm.at[pl.ds(wid * NCHUNK, NCHUNK)], idx_v)
    for k in range(NVREG):
        j, off = k // 8, (k % 8) * LANES
        v = idx_v[j, pl.ds(off, LANES)]
        idx16_v[j, pl.ds(off, LANES)] = lax.shift_right_logical(v, 4)
    copies = []
    for j in range(NCHUNK):
        sl = pl.ds(j * IDX_CHUNK, IDX_CHUNK)
        idx_j = idx_v.at[j]
        idx16_j = idx16_v.at[j]
        copies.append(pltpu.async_copy(emb_hbm.at[idx_j], rows_v.at[sl], sem))
        copies.append(pltpu.async_copy(smap_hbm.at[idx16_j], ms_v.at[sl], sem))
        copies.append(pltpu.async_copy(gmap_hbm.at[idx16_j], mg_v.at[sl], sem))
        copies.append(pltpu.async_copy(grmap_hbm.at[idx16_j], mgr_v.at[sl], sem))
        copies.append(pltpu.async_copy(cmap_hbm.at[idx16_j], mc_v.at[sl], sem))
    for cp in copies:
        cp.wait()
    iota = lax.iota(jnp.int32, LANES)
    for k in range(NVREG):
        j, off = k // 8, (k % 8) * LANES
        lanes = jnp.bitwise_and(idx_v[j, pl.ds(off, LANES)], 15)
        row_ids = iota + (k * LANES)
        for c, mref in ((0, ms_v), (1, mg_v), (2, mgr_v), (3, mc_v)):
            val = plsc.load_gather(mref, [row_ids, lanes])
            plsc.store_scatter(sidx_v, [row_ids, jnp.full((LANES,), c, jnp.int32)],
                               plsc.bitcast(val, jnp.float32))
    out_rows = out_hbm.at[pl.ds(base, BPW)]
    pltpu.sync_copy(rows_v, out_rows.at[:, pl.ds(0, D_ART)])
    pltpu.sync_copy(sidx_v, out_rows.at[:, pl.ds(D_ART, 4)])


BLK = 2048


def _tc_body(art_ref, semb_ref, gemb_ref, gremb_ref, cemb_ref,
             gamma_ref, beta_ref, mean_ref, var_ref, w_ref, out_ref):
    scale = gamma_ref[:] * lax.rsqrt(var_ref[:] + EPS)
    shift = beta_ref[:] - mean_ref[:] * scale

    def onehot_feat(col, emb_ref, ncls):
        idx = lax.bitcast_convert_type(
            art_ref[:, D_ART + col:D_ART + col + 1], jnp.int32)
        oh = (idx == lax.broadcasted_iota(jnp.int32, (1, ncls), 1))
        return jnp.dot(oh.astype(jnp.float32), emb_ref[:],
                       preferred_element_type=jnp.float32)

    xs = onehot_feat(0, semb_ref, 64)
    xg = onehot_feat(1, gemb_ref, 32)
    xgr = onehot_feat(2, gremb_ref, 32)
    xc = onehot_feat(3, cemb_ref, 32)
    x = jnp.concatenate([art_ref[:, :D_ART], xg, xgr, xc, xs], axis=1)
    x = x * scale + shift
    out_ref[:] = jnp.dot(x, w_ref[:], preferred_element_type=jnp.float32)


def _tc_dense(art, semb, gemb, gremb, cemb, gamma, beta, mean, var, w):
    grid = (B // BLK,)
    full = lambda a: pl.BlockSpec(a.shape, lambda i: tuple(0 for _ in a.shape))
    return pl.pallas_call(
        _tc_body,
        grid=grid,
        in_specs=[
            pl.BlockSpec((BLK, 128), lambda i: (i, 0)),
            full(semb), full(gemb), full(gremb), full(cemb),
            full(gamma), full(beta), full(mean), full(var), full(w),
        ],
        out_specs=pl.BlockSpec((BLK, 128), lambda i: (i, 0)),
        out_shape=jax.ShapeDtypeStruct((B, 128), jnp.float32),
    )(art, semb, gemb, gremb, cemb, gamma, beta, mean, var, w)


def kernel(article_id, article_emb, section_map, section_emb, group_map,
           group_emb, graphical_map, graphical_emb, colour_map, colour_emb,
           gamma, beta, moving_mean, moving_var, W):
    ids = article_id.astype(jnp.int32).reshape(B // IDX_CHUNK, IDX_CHUNK)
    art = _sc_gather(
        ids, article_emb,
        section_map.reshape(V // LANES, LANES),
        group_map.reshape(V // LANES, LANES),
        graphical_map.reshape(V // LANES, LANES),
        colour_map.reshape(V // LANES, LANES))
    return _tc_dense(
        art, section_emb, group_emb, graphical_emb, colour_emb,
        gamma.reshape(1, 128), beta.reshape(1, 128),
        moving_mean.reshape(1, 128), moving_var.reshape(1, 128), W)
